# baseline (device time: 73786 ns/iter reference)
import jax
import jax.numpy as jnp
from jax import lax
from jax.experimental import pallas as pl
from jax.experimental.pallas import tpu as pltpu

N_DEV = 4
EPS = 1e-5
BLK = 512
PRE = 2


def kernel(x, gamma):
    m, n_local = x.shape
    n_global = n_local * N_DEV
    G = m // BLK
    tb = BLK // 128
    g2 = gamma.reshape(1, n_local)

    def body(x_hbm, g_ref, out_ref, xv, inv_ref, comm_ref, dma_sems,
             send_sems, recv_sems):
        i = pl.program_id(0)
        my = lax.axis_index("i")

        def in_dma(b):
            slot = b % PRE if isinstance(b, int) else lax.rem(b, PRE)
            return pltpu.make_async_copy(
                x_hbm.at[pl.ds(b * BLK, BLK), :],
                xv.at[b],
                dma_sems.at[slot],
            )

        @pl.when(i == 0)
        def _start():
            barrier = pltpu.get_barrier_semaphore()
            for k in range(1, N_DEV):
                peer = lax.rem(my + k, N_DEV)
                pl.semaphore_signal(
                    barrier, inc=1,
                    device_id=(peer,), device_id_type=pl.DeviceIdType.MESH,
                )
            pl.semaphore_wait(barrier, N_DEV - 1)
            for b in range(PRE):
                in_dma(b).start()

        @pl.when((i > 0) & (i + PRE - 1 < G))
        def _prefetch():
            in_dma(i + PRE - 1).start()

        @pl.when(i < G)
        def _partial():
            in_dma(i).wait()
            x3 = xv[i].reshape(tb, 128, n_local)
            comm_ref[0, i] = jnp.sum(x3 * x3, axis=2)

        @pl.when(i == G - 1)
        def _exchange():
            rdmas = []
            for k in range(1, N_DEV):
                rdma = pltpu.make_async_remote_copy(
                    src_ref=comm_ref.at[0],
                    dst_ref=comm_ref.at[N_DEV - k],
                    send_sem=send_sems.at[k - 1],
                    recv_sem=recv_sems.at[N_DEV - k],
                    device_id=(lax.rem(my + k, N_DEV),),
                    device_id_type=pl.DeviceIdType.MESH,
                )
                rdma.start()
                rdmas.append(rdma)
            for rdma in rdmas:
                rdma.wait()
            total = (
                comm_ref[0] + comm_ref[1] + comm_ref[2] + comm_ref[3]
            )
            inv_ref[:, :, :] = lax.rsqrt(total / n_global + EPS)

        @pl.when(i >= G)
        def _scale():
            b = i - G
            x3 = xv[b].reshape(tb, 128, n_local)
            s3 = inv_ref[b].reshape(tb, 128, 1)
            gw = g_ref[:, :].reshape(1, 1, n_local)
            out_ref[:, :] = (x3 * s3 * gw).reshape(BLK, n_local)

    return pl.pallas_call(
        body,
        grid=(2 * G,),
        out_shape=jax.ShapeDtypeStruct((m, n_local), x.dtype),
        in_specs=[
            pl.BlockSpec(memory_space=pltpu.MemorySpace.HBM),
            pl.BlockSpec((1, n_local), lambda i: (0, 0)),
        ],
        out_specs=pl.BlockSpec(
            (BLK, n_local), lambda i: (jnp.maximum(i - G, 0), 0)
        ),
        scratch_shapes=[
            pltpu.VMEM((G, BLK, n_local), jnp.float32),
            pltpu.VMEM((G, tb, 128), jnp.float32),
            pltpu.VMEM((N_DEV, G, tb, 128), jnp.float32),
            pltpu.SemaphoreType.DMA((PRE,)),
            pltpu.SemaphoreType.DMA((N_DEV - 1,)),
            pltpu.SemaphoreType.DMA((N_DEV,)),
        ],
        compiler_params=pltpu.CompilerParams(
            collective_id=0,
            vmem_limit_bytes=62 * 1024 * 1024,
        ),
    )(x, g2)


# device time: 53091 ns/iter; 1.3898x vs baseline; 1.3898x over previous
import jax
import jax.numpy as jnp
from jax import lax
from jax.experimental import pallas as pl
from jax.experimental.pallas import tpu as pltpu

N_DEV = 4
EPS = 1e-5
BLK = 1024
LAG = 2


def kernel(x, gamma):
    m, n_local = x.shape
    n_global = n_local * N_DEV
    rows = m // 128
    G = m // BLK
    tb = BLK // 128
    g2 = gamma.reshape(1, n_local)

    def body_a(x_ref, inv_ref, comm_ref, send_sems, recv_sems):
        i = pl.program_id(0)
        my = lax.axis_index("i")

        def slab_rdma(k, b):
            return pltpu.make_async_remote_copy(
                src_ref=comm_ref.at[0, b],
                dst_ref=comm_ref.at[N_DEV - k, b],
                send_sem=send_sems.at[k - 1, b],
                recv_sem=recv_sems.at[N_DEV - k, b],
                device_id=(lax.rem(my + k, N_DEV),),
                device_id_type=pl.DeviceIdType.MESH,
            )

        @pl.when(i == 0)
        def _barrier():
            barrier = pltpu.get_barrier_semaphore()
            for k in range(1, N_DEV):
                pl.semaphore_signal(
                    barrier, inc=1,
                    device_id=(lax.rem(my + k, N_DEV),),
                    device_id_type=pl.DeviceIdType.MESH,
                )
            pl.semaphore_wait(barrier, N_DEV - 1)

        @pl.when(i < G)
        def _partial():
            x3 = x_ref[:, :].reshape(tb, 128, n_local)
            comm_ref[0, i] = jnp.sum(x3 * x3, axis=2)
            for k in range(1, N_DEV):
                slab_rdma(k, i).start()

        @pl.when(i >= LAG)
        def _reduce():
            j = i - LAG
            for k in range(1, N_DEV):
                slab_rdma(k, j).wait()
            total = (
                comm_ref[0, j] + comm_ref[1, j]
                + comm_ref[2, j] + comm_ref[3, j]
            )
            inv_ref[:, :] = lax.rsqrt(total / n_global + EPS)

    inv = pl.pallas_call(
        body_a,
        grid=(G + LAG,),
        out_shape=jax.ShapeDtypeStruct((rows, 128), jnp.float32),
        in_specs=[
            pl.BlockSpec((BLK, n_local), lambda i: (jnp.minimum(i, G - 1), 0)),
        ],
        out_specs=pl.BlockSpec((tb, 128), lambda i: (jnp.maximum(i - LAG, 0), 0)),
        scratch_shapes=[
            pltpu.VMEM((N_DEV, G, tb, 128), jnp.float32),
            pltpu.SemaphoreType.DMA((N_DEV - 1, G)),
            pltpu.SemaphoreType.DMA((N_DEV, G)),
        ],
        compiler_params=pltpu.CompilerParams(
            collective_id=0,
            vmem_limit_bytes=48 * 1024 * 1024,
        ),
    )(x)

    def body_b(x_ref, inv_ref, g_ref, out_ref):
        x3 = x_ref[:, :].reshape(tb, 128, n_local)
        s3 = inv_ref[:, :].reshape(tb, 128, 1)
        out_ref[:, :] = (x3 * s3 * g_ref[:, :]).reshape(BLK, n_local)

    return pl.pallas_call(
        body_b,
        grid=(G,),
        out_shape=jax.ShapeDtypeStruct((m, n_local), x.dtype),
        in_specs=[
            pl.BlockSpec((BLK, n_local), lambda i: (i, 0)),
            pl.BlockSpec((tb, 128), lambda i: (i, 0)),
            pl.BlockSpec((1, n_local), lambda i: (0, 0)),
        ],
        out_specs=pl.BlockSpec((BLK, n_local), lambda i: (i, 0)),
        compiler_params=pltpu.CompilerParams(
            vmem_limit_bytes=48 * 1024 * 1024,
        ),
    )(x, inv, g2)
